# Initial kernel scaffold; baseline (speedup 1.0000x reference)
import jax
import jax.numpy as jnp
from jax.experimental import pallas as pl


def kernel(nodes, features, edges, W1, W2, W3, W4, W5, W6, W7, W_lin):
    raise NotImplementedError("write your pallas kernel here")


# speedup vs baseline: 1.2990x; 1.2990x over previous
import functools

import jax
import jax.numpy as jnp
from jax import lax
from jax.experimental import pallas as pl
from jax.experimental.pallas import tpu as pltpu
from jax.experimental.pallas import tpu_sc as plsc

NC, NS, L = 2, 16, 16
NW = NC * NS
E = 1600000
EPW = E // NW
CHUNK = 2000
NV = CHUNK // L
NCH = EPW // CHUNK

_MESH = plsc.VectorSubcoreMesh(
    core_axis_name="c", subcore_axis_name="s", num_cores=NC, num_subcores=NS)


def _scatmax(acc, d, v):
    def cond(carry):
        return jnp.logical_not(carry[1])

    def body(carry):
        vv, _ = carry
        cur = plsc.load_gather(acc, (d,))
        m = jnp.maximum(cur, vv)
        plsc.store_scatter(acc, (d,), m)
        chk = plsc.load_gather(acc, (d,))
        return m, jnp.all(chk >= m)

    lax.while_loop(cond, body, (v, jnp.array(False)))


def _segmax_small_body(n, s_hbm, d_hbm, tab_hbm, out_hbm,
                       tab_v, acc_v, s_v, d_v, shared, mrg_v, tmp_v):
    cid = lax.axis_index("c")
    sid = lax.axis_index("s")
    wid = cid * NS + sid
    base = wid * EPW
    sl = n // NS

    pltpu.sync_copy(tab_hbm, tab_v)
    pltpu.sync_copy(tab_hbm, acc_v)

    def chunk_body(ch, _):
        off = base + ch * CHUNK
        pltpu.sync_copy(s_hbm.at[pl.ds(off, CHUNK)], s_v)
        pltpu.sync_copy(d_hbm.at[pl.ds(off, CHUNK)], d_v)

        def vec_body(i, _):
            sv = s_v[pl.ds(i * L, L)]
            dv = d_v[pl.ds(i * L, L)]
            vals = plsc.load_gather(tab_v, (sv,))
            _scatmax(acc_v, dv, vals)
            return 0

        return lax.fori_loop(0, NV, vec_body, 0)

    lax.fori_loop(0, NCH, chunk_body, 0)

    pltpu.sync_copy(acc_v, shared.at[sid])
    plsc.subcore_barrier()
    off = sid * sl
    pltpu.sync_copy(shared.at[0, pl.ds(off, sl)], mrg_v)
    for k in range(1, NS):
        pltpu.sync_copy(shared.at[k, pl.ds(off, sl)], tmp_v)

        def max_body(i, _):
            mrg_v[pl.ds(i * L, L)] = jnp.maximum(
                mrg_v[pl.ds(i * L, L)], tmp_v[pl.ds(i * L, L)])
            return 0

        lax.fori_loop(0, sl // L, max_body, 0)
    pltpu.sync_copy(mrg_v, out_hbm.at[cid, pl.ds(off, sl)])


def _segmax_small(n):
    sl = n // NS
    return pl.kernel(
        functools.partial(_segmax_small_body, n),
        out_type=jax.ShapeDtypeStruct((NC, n), jnp.float32),
        mesh=_MESH,
        compiler_params=pltpu.CompilerParams(needs_layout_passes=False),
        scratch_types=[
            pltpu.VMEM((n,), jnp.float32),
            pltpu.VMEM((n,), jnp.float32),
            pltpu.VMEM((CHUNK,), jnp.int32),
            pltpu.VMEM((CHUNK,), jnp.int32),
            pltpu.VMEM_SHARED((NS, n), jnp.float32),
            pltpu.VMEM((sl,), jnp.float32),
            pltpu.VMEM((sl,), jnp.float32),
        ],
    )


def _proj_body(pf_ref, w1_ref, w2_ref, w3_ref, w4_ref, w5_ref, w6_ref, w7_ref, wl_ref, out_ref):
    hp = functools.partial(jnp.dot, precision=lax.Precision.HIGHEST)
    w = jax.nn.relu(w1_ref[...])
    w = jax.nn.relu(hp(w, w2_ref[...]))
    w = jax.nn.relu(hp(w, w3_ref[...]))
    w = jax.nn.relu(hp(w, w4_ref[...]))
    w = jax.nn.relu(hp(w, w5_ref[...]))
    w = jax.nn.relu(hp(w, w6_ref[...]))
    c7 = jax.nn.relu(hp(w, w7_ref[...]))
    scaled = pf_ref[...] * c7
    acc = jnp.zeros((1, 100), jnp.float32)
    for g in range(64):
        acc = acc + hp(scaled[g:g + 1, :], wl_ref[g])
    out_ref[...] = acc


def _project(pf, W1, W2, W3, W4, W5, W6, W7, W_lin):
    return pl.pallas_call(
        _proj_body,
        out_shape=jax.ShapeDtypeStruct((1, 100), jnp.float32),
    )(pf.reshape(64, 1), W1, W2, W3, W4, W5, W6, W7, W_lin.reshape(64, 128, 100))


def _segmax(vals, idx, n):
    out = jnp.zeros((n,), jnp.float32)
    return out.at[idx].max(vals, mode="promise_in_bounds")


def kernel(nodes, features, edges, W1, W2, W3, W4, W5, W6, W7, W_lin):
    feats = features[:, 0]
    src, dst = edges[0], edges[1]
    n = nodes.shape[0]

    cid1 = ((nodes[:, 0] // 4) * 64 + (nodes[:, 1] // 4)) * 64 + (nodes[:, 2] // 4)
    A1 = _segmax(feats[src], dst, n)
    A2 = _segmax(A1[src], dst, n)
    P1 = _segmax(A2, cid1, 64 * 64 * 64)
    s1, d1 = cid1[src], cid1[dst]
    A3 = jnp.maximum(P1, _segmax(P1[s1], d1, 64 * 64 * 64))
    c = jnp.arange(64 * 64 * 64, dtype=jnp.int32)
    cid2_tab = ((c // (64 * 64)) // 4 * 16 + ((c // 64) % 64) // 4) * 16 + (c % 64) // 4
    P2 = jnp.maximum(
        jnp.max(A3.reshape(16, 4, 16, 4, 16, 4), axis=(1, 3, 5)).reshape(-1),
        _segmax(A3[s1], cid2_tab[d1], 4096))
    s2, d2 = cid2_tab[s1], cid2_tab[d1]

    conv = _segmax_small(4096)
    A5 = jnp.max(conv(s2, d2, P2), axis=0)
    A6 = jnp.max(conv(s2, d2, A5), axis=0)
    A7 = jnp.max(conv(s2, d2, A6), axis=0)

    Pf = jnp.max(A7.reshape(4, 4, 4, 4, 4, 4), axis=(1, 3, 5)).reshape(-1)
    return _project(Pf, W1, W2, W3, W4, W5, W6, W7, W_lin)


# speedup vs baseline: 47.5436x; 36.5994x over previous
import functools

import jax
import jax.numpy as jnp
from jax import lax
from jax.experimental import pallas as pl
from jax.experimental.pallas import tpu as pltpu
from jax.experimental.pallas import tpu_sc as plsc

NC, NS, L = 2, 16, 16
NW = NC * NS
E = 1600000
N = 100000
NP = 100352
N1 = 64 * 64 * 64
N2 = 16 * 16 * 16
CHUNK = 2000
NRANGE = 4
RSZ = N1 // NRANGE

_MESH = plsc.VectorSubcoreMesh(
    core_axis_name="c", subcore_axis_name="s", num_cores=NC, num_subcores=NS)
_PARAMS = pltpu.CompilerParams(needs_layout_passes=False)

_i32 = jnp.int32
_f32 = jnp.float32


def _ids():
    return lax.axis_index("c"), lax.axis_index("s")


def _vloop(ref_a, ref_b, nvec):
    def body(i, _):
        ref_a[pl.ds(i * L, L)] = jnp.maximum(
            ref_a[pl.ds(i * L, L)], ref_b[pl.ds(i * L, L)])
        return 0
    lax.fori_loop(0, nvec, body, 0)


def _fill_zero(ref, nvec):
    z = jnp.zeros((L,), _f32)
    def body(i, _):
        ref[pl.ds(i * L, L)] = z
        return 0
    lax.fori_loop(0, nvec, body, 0)


def _scatmax_masked(acc, d, v, msk):
    dl = jnp.where(msk, d, 0)
    vv = jnp.where(msk, v, -1.0)

    def cond(carry):
        return carry[1]

    def body(carry):
        pend, _ = carry
        cur = plsc.load_gather(acc, (dl,), mask=pend)
        m = jnp.where(pend, jnp.maximum(cur, vv), -1.0)
        plsc.store_scatter(acc, (dl,), m, mask=pend)
        chk = plsc.load_gather(acc, (dl,), mask=pend)
        npend = jnp.logical_and(pend, chk < m)
        return npend, jnp.any(npend)

    lax.while_loop(cond, body, (msk, jnp.array(True)))


def _scatmax(acc, d, v):
    _scatmax_masked(acc, d, v, jnp.ones((L,), jnp.bool_))


def _stage_spmem(tab_hbm, nrows, n, spm, sid, buf_a, buf_b):
    sl = n // NS
    off = sid * sl
    pltpu.sync_copy(tab_hbm.at[pl.ds(off, sl)], buf_a)
    for k in range(1, nrows):
        pltpu.sync_copy(tab_hbm.at[pl.ds(k * n + off, sl)], buf_b)
        _vloop(buf_a, buf_b, sl // L)
    pltpu.sync_copy(buf_a, spm.at[pl.ds(off, sl)])
    plsc.subcore_barrier()


def _stage_vmem(tab_hbm, nrows, n, dst_v, tmp_v):
    pltpu.sync_copy(tab_hbm.at[pl.ds(0, n)], dst_v)
    for k in range(1, nrows):
        pltpu.sync_copy(tab_hbm.at[pl.ds(k * n, n)], tmp_v)
        _vloop(dst_v, tmp_v, n // L)


def _cid2_of(c):
    x2 = lax.shift_right_logical(c, 14)
    y2 = jnp.bitwise_and(lax.shift_right_logical(c, 8), 15)
    z2 = jnp.bitwise_and(lax.shift_right_logical(c, 2), 15)
    return (x2 * 256 + y2 * 16) + z2


def _cid_body(nx_hbm, ny_hbm, nz_hbm, out_hbm, nx_v, ny_v, nz_v, o_v):
    cid, sid = _ids()
    w = cid * NS + sid
    npw = NP // NW
    base = w * npw
    pltpu.sync_copy(nx_hbm.at[pl.ds(base, npw)], nx_v)
    pltpu.sync_copy(ny_hbm.at[pl.ds(base, npw)], ny_v)
    pltpu.sync_copy(nz_hbm.at[pl.ds(base, npw)], nz_v)

    def body(i, _):
        x = lax.shift_right_logical(nx_v[pl.ds(i * L, L)], 2)
        y = lax.shift_right_logical(ny_v[pl.ds(i * L, L)], 2)
        z = lax.shift_right_logical(nz_v[pl.ds(i * L, L)], 2)
        o_v[pl.ds(i * L, L)] = (x * 64 + y) * 64 + z
        return 0

    lax.fori_loop(0, npw // L, body, 0)
    pltpu.sync_copy(o_v, out_hbm.at[pl.ds(base, npw)])


_cid_kernel = pl.kernel(
    _cid_body,
    out_type=jax.ShapeDtypeStruct((NP,), _i32),
    mesh=_MESH, compiler_params=_PARAMS,
    scratch_types=[
        pltpu.VMEM((NP // NW,), _i32),
        pltpu.VMEM((NP // NW,), _i32),
        pltpu.VMEM((NP // NW,), _i32),
        pltpu.VMEM((NP // NW,), _i32),
    ],
)


def _rowmax_body(n, nrows, in_hbm, out_hbm, a_v, b_v):
    cid, sid = _ids()
    w = cid * NS + sid
    sl = n // NW
    off = w * sl
    pltpu.sync_copy(in_hbm.at[pl.ds(off, sl)], a_v)
    for k in range(1, nrows):
        pltpu.sync_copy(in_hbm.at[pl.ds(k * n + off, sl)], b_v)
        _vloop(a_v, b_v, sl // L)
    pltpu.sync_copy(a_v, out_hbm.at[pl.ds(off, sl)])


def _rowmax(n, nrows):
    return pl.kernel(
        functools.partial(_rowmax_body, n, nrows),
        out_type=jax.ShapeDtypeStruct((n,), _f32),
        mesh=_MESH, compiler_params=_PARAMS,
        scratch_types=[
            pltpu.VMEM((n // NW,), _f32),
            pltpu.VMEM((n // NW,), _f32),
        ])


def _nodepass_body(emit_s1d1, src_hbm, dst_hbm, tab_hbm, *refs):
    if emit_s1d1:
        (cid1_hbm, out_hbm, s1_hbm, d1_hbm,
         acc_v, s_v, d_v, vals_v, s1_v, d1_v, sem) = refs
    else:
        (out_hbm,
         acc_v, s_v, d_v, vals_v, sem) = refs
    cid, sid = _ids()
    w = cid * NS + sid
    epw = E // NW
    base = w * epw

    _fill_zero(acc_v, NP // L)

    def chunk(chi, _):
        off = base + chi * CHUNK
        pltpu.sync_copy(src_hbm.at[pl.ds(off, CHUNK)], s_v)
        pltpu.sync_copy(dst_hbm.at[pl.ds(off, CHUNK)], d_v)
        pltpu.async_copy(tab_hbm.at[s_v], vals_v, sem).wait()
        if emit_s1d1:
            pltpu.async_copy(cid1_hbm.at[s_v], s1_v, sem).wait()
            pltpu.async_copy(cid1_hbm.at[d_v], d1_v, sem).wait()
            pltpu.sync_copy(s1_v, s1_hbm.at[pl.ds(off, CHUNK)])
            pltpu.sync_copy(d1_v, d1_hbm.at[pl.ds(off, CHUNK)])

        def vec(i, _):
            dv = d_v[pl.ds(i * L, L)]
            vv = vals_v[pl.ds(i * L, L)]
            _scatmax(acc_v, dv, vv)
            return 0

        return lax.fori_loop(0, CHUNK // L, vec, 0)

    lax.fori_loop(0, E // NW // CHUNK, chunk, 0)
    pltpu.sync_copy(acc_v, out_hbm.at[pl.ds(w * NP, NP)])


def _nodepass(emit_s1d1):
    out_types = jax.ShapeDtypeStruct((NW * NP,), _f32)
    scratch = [
        pltpu.VMEM((NP,), _f32),
        pltpu.VMEM((CHUNK,), _i32),
        pltpu.VMEM((CHUNK,), _i32),
        pltpu.VMEM((CHUNK,), _f32),
    ]
    if emit_s1d1:
        out_types = (out_types,
                     jax.ShapeDtypeStruct((E,), _i32),
                     jax.ShapeDtypeStruct((E,), _i32))
        scratch += [pltpu.VMEM((CHUNK,), _i32), pltpu.VMEM((CHUNK,), _i32)]
    scratch += [pltpu.SemaphoreType.DMA]
    return pl.kernel(
        functools.partial(_nodepass_body, emit_s1d1),
        out_type=out_types, mesh=_MESH, compiler_params=_PARAMS,
        scratch_types=scratch)


def _pool_body(a2_hbm, cid1_hbm, p1_hbm, pp_hbm,
               acc_v, pp_v, c_v, a_v):
    cid, sid = _ids()
    rg = sid // 4
    li = sid % 4
    part = cid * 4 + li
    npw = NP // 8
    base = part * npw
    lo = rg * RSZ
    csz = 1568
    nch = npw // csz

    _fill_zero(acc_v, RSZ // L)
    _fill_zero(pp_v, N2 // L)

    def chunk(chi, _):
        off = base + chi * csz
        pltpu.sync_copy(cid1_hbm.at[pl.ds(off, csz)], c_v)
        pltpu.sync_copy(a2_hbm.at[pl.ds(off, csz)], a_v)

        def vec(i, _):
            cc = c_v[pl.ds(i * L, L)]
            vv = a_v[pl.ds(i * L, L)]
            _scatmax(pp_v, _cid2_of(cc), vv)
            msk = jnp.logical_and(cc >= lo, cc < lo + RSZ)
            _scatmax_masked(acc_v, cc - lo, vv, msk)
            return 0

        return lax.fori_loop(0, csz // L, vec, 0)

    lax.fori_loop(0, nch, chunk, 0)
    pltpu.sync_copy(acc_v, p1_hbm.at[pl.ds((cid * 4 + li) * N1 + lo, RSZ)])
    pltpu.sync_copy(pp_v, pp_hbm.at[pl.ds((cid * NS + sid) * N2, N2)])


_pool_kernel = pl.kernel(
    _pool_body,
    out_type=(jax.ShapeDtypeStruct((8 * N1,), _f32),
              jax.ShapeDtypeStruct((NW * N2,), _f32)),
    mesh=_MESH, compiler_params=_PARAMS,
    scratch_types=[
        pltpu.VMEM((RSZ,), _f32),
        pltpu.VMEM((N2,), _f32),
        pltpu.VMEM((1568,), _i32),
        pltpu.VMEM((1568,), _f32),
    ])


def _a3_body(s1_hbm, d1_hbm, p1_hbm, out_hbm,
             acc_v, s_v, d_v, vals_v, sem):
    cid, sid = _ids()
    rg = sid // 4
    li = sid % 4
    part = cid * 4 + li
    epr = E // 8
    base = part * epr
    lo = rg * RSZ

    pltpu.sync_copy(p1_hbm.at[pl.ds(lo, RSZ)], acc_v)

    def chunk(chi, _):
        off = base + chi * CHUNK
        pltpu.sync_copy(s1_hbm.at[pl.ds(off, CHUNK)], s_v)
        pltpu.sync_copy(d1_hbm.at[pl.ds(off, CHUNK)], d_v)
        pltpu.async_copy(p1_hbm.at[s_v], vals_v, sem).wait()

        def vec(i, _):
            dv = d_v[pl.ds(i * L, L)]
            vv = vals_v[pl.ds(i * L, L)]
            msk = jnp.logical_and(dv >= lo, dv < lo + RSZ)
            _scatmax_masked(acc_v, dv - lo, vv, msk)
            return 0

        return lax.fori_loop(0, CHUNK // L, vec, 0)

    lax.fori_loop(0, epr // CHUNK, chunk, 0)
    pltpu.sync_copy(acc_v, out_hbm.at[pl.ds((cid * 4 + li) * N1 + lo, RSZ)])


_a3_kernel = pl.kernel(
    _a3_body,
    out_type=jax.ShapeDtypeStruct((8 * N1,), _f32),
    mesh=_MESH, compiler_params=_PARAMS,
    scratch_types=[
        pltpu.VMEM((RSZ,), _f32),
        pltpu.VMEM((CHUNK,), _i32),
        pltpu.VMEM((CHUNK,), _i32),
        pltpu.VMEM((CHUNK,), _f32),
        pltpu.SemaphoreType.DMA,
    ])


def _p2_body(s1_hbm, d1_hbm, a3_hbm, pp_hbm, out_hbm,
             acc_v, s_v, d_v, vals_v, tmp_v, sem):
    cid, sid = _ids()
    w = cid * NS + sid
    epw = E // NW
    base = w * epw

    _stage_vmem(pp_hbm, NW, N2, acc_v, tmp_v)

    def chunk(chi, _):
        off = base + chi * CHUNK
        pltpu.sync_copy(s1_hbm.at[pl.ds(off, CHUNK)], s_v)
        pltpu.sync_copy(d1_hbm.at[pl.ds(off, CHUNK)], d_v)
        pltpu.async_copy(a3_hbm.at[s_v], vals_v, sem).wait()

        def vec(i, _):
            dv = _cid2_of(d_v[pl.ds(i * L, L)])
            vv = vals_v[pl.ds(i * L, L)]
            _scatmax(acc_v, dv, vv)
            return 0

        return lax.fori_loop(0, CHUNK // L, vec, 0)

    lax.fori_loop(0, epw // CHUNK, chunk, 0)
    pltpu.sync_copy(acc_v, out_hbm.at[pl.ds(w * N2, N2)])


_p2_kernel = pl.kernel(
    _p2_body,
    out_type=jax.ShapeDtypeStruct((NW * N2,), _f32),
    mesh=_MESH, compiler_params=_PARAMS,
    scratch_types=[
        pltpu.VMEM((N2,), _f32),
        pltpu.VMEM((CHUNK,), _i32),
        pltpu.VMEM((CHUNK,), _i32),
        pltpu.VMEM((CHUNK,), _f32),
        pltpu.VMEM((N2,), _f32),
        pltpu.SemaphoreType.DMA,
    ])


def _conv2_body(s1_hbm, d1_hbm, tab_hbm, out_hbm,
                tab_v, acc_v, tmp_v, s_v, d_v):
    cid, sid = _ids()
    w = cid * NS + sid
    epw = E // NW
    base = w * epw

    _stage_vmem(tab_hbm, NW, N2, tab_v, tmp_v)

    def initv(i, _):
        acc_v[pl.ds(i * L, L)] = tab_v[pl.ds(i * L, L)]
        return 0
    lax.fori_loop(0, N2 // L, initv, 0)

    def chunk(chi, _):
        off = base + chi * CHUNK
        pltpu.sync_copy(s1_hbm.at[pl.ds(off, CHUNK)], s_v)
        pltpu.sync_copy(d1_hbm.at[pl.ds(off, CHUNK)], d_v)

        def vec(i, _):
            sv = _cid2_of(s_v[pl.ds(i * L, L)])
            dv = _cid2_of(d_v[pl.ds(i * L, L)])
            vals = plsc.load_gather(tab_v, (sv,))
            _scatmax(acc_v, dv, vals)
            return 0

        return lax.fori_loop(0, CHUNK // L, vec, 0)

    lax.fori_loop(0, epw // CHUNK, chunk, 0)
    pltpu.sync_copy(acc_v, out_hbm.at[pl.ds(w * N2, N2)])


_conv2_kernel = pl.kernel(
    _conv2_body,
    out_type=jax.ShapeDtypeStruct((NW * N2,), _f32),
    mesh=_MESH, compiler_params=_PARAMS,
    scratch_types=[
        pltpu.VMEM((N2,), _f32),
        pltpu.VMEM((N2,), _f32),
        pltpu.VMEM((N2,), _f32),
        pltpu.VMEM((CHUNK,), _i32),
        pltpu.VMEM((CHUNK,), _i32),
    ])


def _pf_body(a7_hbm, out_hbm, a_v, b_v, pf_v):
    cid, sid = _ids()

    @pl.when(jnp.logical_and(cid == 0, sid == 0))
    def _():
        _stage_vmem(a7_hbm, NW, N2, a_v, b_v)
        _fill_zero(pf_v, 64 // L)
        zi = lax.shift_right_logical(lax.iota(_i32, L), 2)

        def vec(i, _):
            x = lax.shift_right_logical(i, 4)
            y = jnp.bitwise_and(i, 15)
            g0 = lax.shift_right_logical(x, 2) * 16 + lax.shift_right_logical(y, 2) * 4
            gidx = zi + g0
            _scatmax(pf_v, gidx, a_v[pl.ds(i * L, L)])
            return 0

        lax.fori_loop(0, N2 // L, vec, 0)
        pltpu.sync_copy(pf_v, out_hbm)


_pf_kernel = pl.kernel(
    _pf_body,
    out_type=jax.ShapeDtypeStruct((64,), _f32),
    mesh=_MESH, compiler_params=_PARAMS,
    scratch_types=[
        pltpu.VMEM((N2,), _f32),
        pltpu.VMEM((N2,), _f32),
        pltpu.VMEM((64,), _f32),
    ])


def _proj_body(pf_ref, w1_ref, w2_ref, w3_ref, w4_ref, w5_ref, w6_ref, w7_ref, wl_ref, out_ref):
    hp = functools.partial(jnp.dot, precision=lax.Precision.HIGHEST)
    w = jax.nn.relu(w1_ref[...])
    w = jax.nn.relu(hp(w, w2_ref[...]))
    w = jax.nn.relu(hp(w, w3_ref[...]))
    w = jax.nn.relu(hp(w, w4_ref[...]))
    w = jax.nn.relu(hp(w, w5_ref[...]))
    w = jax.nn.relu(hp(w, w6_ref[...]))
    c7 = jax.nn.relu(hp(w, w7_ref[...]))
    scaled = pf_ref[...] * c7
    acc = jnp.zeros((1, 100), _f32)
    for g in range(64):
        acc = acc + hp(scaled[g:g + 1, :], wl_ref[g])
    out_ref[...] = acc


def _project(pf, W1, W2, W3, W4, W5, W6, W7, W_lin):
    return pl.pallas_call(
        _proj_body,
        out_shape=jax.ShapeDtypeStruct((1, 100), jnp.float32),
    )(pf.reshape(64, 1), W1, W2, W3, W4, W5, W6, W7, W_lin.reshape(64, 128, 100))


def kernel(nodes, features, edges, W1, W2, W3, W4, W5, W6, W7, W_lin):
    feats = jnp.pad(features[:, 0], (0, NP - N))
    nx = jnp.pad(nodes[:, 0], (0, NP - N))
    ny = jnp.pad(nodes[:, 1], (0, NP - N))
    nz = jnp.pad(nodes[:, 2], (0, NP - N))
    src, dst = edges[0], edges[1]

    cid1 = _cid_kernel(nx, ny, nz)
    A1 = _rowmax(NP, NW)(_nodepass(False)(src, dst, feats))
    A2r, s1, d1 = _nodepass(True)(src, dst, A1, cid1)
    A2 = _rowmax(NP, NW)(A2r)
    P1, Ppool = _pool_kernel(A2, cid1)
    P1 = _rowmax(N1, 8)(P1)
    A3 = _rowmax(N1, 8)(_a3_kernel(s1, d1, P1))
    P2 = _p2_kernel(s1, d1, A3, Ppool)
    A5 = _conv2_kernel(s1, d1, P2)
    A6 = _conv2_kernel(s1, d1, A5)
    A7 = _conv2_kernel(s1, d1, A6)
    Pf = _pf_kernel(A7)
    return _project(Pf, W1, W2, W3, W4, W5, W6, W7, W_lin)
